# baseline (speedup 1.0000x reference)
import functools

import jax
import jax.numpy as jnp
from jax import lax
from jax.experimental import pallas as pl
from jax.experimental.pallas import tpu as pltpu
from jax.experimental.pallas import tpu_sc as plsc

EMBEDDING_DIM = 32
NUM_CORES = 2
NUM_SUBCORES = 16
NUM_WORKERS = NUM_CORES * NUM_SUBCORES
CHUNK_B = 16
NBUF = 2
LANES = 16


@functools.partial(jax.jit, static_argnums=(2, 3))
def _gather_rows(table, x, batch, hist):
    mesh = plsc.VectorSubcoreMesh(core_axis_name="c", subcore_axis_name="s")
    bpw = batch // NUM_WORKERS
    n_chunks = bpw // CHUNK_B
    d8 = EMBEDDING_DIM // 8
    bt = batch // 128

    @functools.partial(
        pl.kernel,
        mesh=mesh,
        out_type=jax.ShapeDtypeStruct((hist, d8, bt, 8, 128), jnp.float32),
        scratch_types=[
            pltpu.VMEM((bpw, hist), jnp.int32),
            pltpu.VMEM((CHUNK_B, hist, EMBEDDING_DIM), jnp.float32),
            pltpu.VMEM((CHUNK_B, hist, EMBEDDING_DIM), jnp.float32),
            pltpu.VMEM((hist, d8, 8, CHUNK_B), jnp.float32),
            pltpu.VMEM((hist, d8, 8, CHUNK_B), jnp.float32),
            pltpu.SemaphoreType.DMA,
            pltpu.SemaphoreType.DMA,
            pltpu.SemaphoreType.DMA,
            pltpu.SemaphoreType.DMA,
        ],
        compiler_params=pltpu.CompilerParams(use_tc_tiling_on_sc=False, needs_layout_passes=False),
    )
    def k(table_hbm, x_hbm, out_hbm, idx_v, rows0, rows1, t0, t1, g0, g1, w0, w1):
        wid = lax.axis_index("s") * NUM_CORES + lax.axis_index("c")
        base = wid * bpw
        rows = (rows0, rows1)
        tbuf = (t0, t1)
        gsem = (g0, g1)
        wsem = (w0, w1)

        pltpu.sync_copy(x_hbm.at[pl.ds(base, bpw)], idx_v)

        iot = lax.iota(jnp.int32, LANES)

        def row_gather_desc(i, j, b):
            return pltpu.make_async_copy(
                table_hbm.at[idx_v.at[i * CHUNK_B + j]],
                rows[b].at[j],
                gsem[b],
            )

        def fire_gathers(i, b):
            def fj(j, c):
                row_gather_desc(i, j, b).start()
                return c

            lax.fori_loop(0, CHUNK_B, fj, 0)

        def drain_gathers(i, b):
            def fj(j, c):
                row_gather_desc(i, j, b).wait()
                return c

            lax.fori_loop(0, CHUNK_B, fj, 0)

        def transpose_chunk(b):
            rflat = rows[b]
            tflat = tbuf[b]

            def fh(h, c):
                hv = jnp.full((LANES,), h, jnp.int32)
                for d in range(EMBEDDING_DIM):
                    dv = jnp.full((LANES,), d, jnp.int32)
                    x16 = plsc.load_gather(rflat, [iot, hv, dv])
                    tflat[h, d // 8, d % 8, pl.ds(0, LANES)] = x16
                return c

            lax.fori_loop(0, hist, fh, 0)

        def write_desc(i, b):
            gb = base + i * CHUNK_B
            tj = gb // 128
            bl0 = gb % 128
            return pltpu.make_async_copy(
                tbuf[b],
                out_hbm.at[pl.ds(0, hist), pl.ds(0, d8), tj, pl.ds(0, 8),
                           pl.ds(bl0, CHUNK_B)],
                wsem[b],
            )

        for b in range(NBUF):
            fire_gathers(b, b)

        def body(g, carry):
            for b in range(NBUF):
                i = g * NBUF + b
                drain_gathers(i, b)
                transpose_chunk(b)
                write_desc(i, b).start()
                write_desc(i, b).wait()
                fire_gathers(i + NBUF, b)
            return carry

        lax.fori_loop(0, n_chunks // NBUF - 1, body, 0)

        for b in range(NBUF):
            i = n_chunks - NBUF + b
            drain_gathers(i, b)
            transpose_chunk(b)
            write_desc(i, b).start()
        for b in range(NBUF):
            i = n_chunks - NBUF + b
            write_desc(i, b).wait()

    return k(table, x)


@jax.jit
def _relayout_table(tT, app):
    mesh = plsc.VectorSubcoreMesh(core_axis_name="c", subcore_axis_name="s")
    d, v = tT.shape
    BK = 512
    n_full = v // BK
    n_main = n_full // NUM_WORKERS
    n_tail = n_full % NUM_WORKERS

    @functools.partial(
        pl.kernel,
        mesh=mesh,
        out_type=jax.ShapeDtypeStruct((v // 4, 128), jnp.float32),
        scratch_types=[
            pltpu.VMEM((d, BK), jnp.float32),
            pltpu.VMEM((d, BK), jnp.float32),
            pltpu.VMEM((BK // 4, 128), jnp.float32),
            pltpu.VMEM((BK // 4, 128), jnp.float32),
            pltpu.SemaphoreType.DMA,
            pltpu.SemaphoreType.DMA,
            pltpu.SemaphoreType.DMA,
            pltpu.SemaphoreType.DMA,
        ],
        compiler_params=pltpu.CompilerParams(
            use_tc_tiling_on_sc=True, needs_layout_passes=False
        ),
    )
    def ka(tT_hbm, app_hbm, rm_hbm, v0, v1, o0, o1, gi0, gi1, go0, go1):
        wid = lax.axis_index("s") * NUM_CORES + lax.axis_index("c")
        vbuf = (v0, v1)
        obuf = (o0, o1)
        isem = (gi0, gi1)
        osem = (go0, go1)

        iot = lax.iota(jnp.int32, LANES)
        io4 = iot // 4
        iom = lax.rem(iot, 4) * 32

        def tc_of(it):
            return jnp.where(it < n_main, wid + it * NUM_WORKERS,
                             n_main * NUM_WORKERS + wid)

        def n_iters():
            return n_main + jnp.where(wid < n_tail, 1, 0)

        def in_desc(tc, b):
            return pltpu.make_async_copy(
                tT_hbm.at[pl.ds(0, d), pl.ds(tc * BK, BK)], vbuf[b], isem[b]
            )

        def out_desc(tc, b):
            return pltpu.make_async_copy(
                obuf[b], rm_hbm.at[pl.ds(tc * (BK // 4), BK // 4)], osem[b]
            )

        def transpose_tc(b):
            def fd(dd, carry):
                colv = iom + dd
                for c in range(0, BK, LANES):
                    qv = io4 + (c // 4)
                    x16 = vbuf[b][dd, pl.ds(c, LANES)]
                    plsc.store_scatter(obuf[b], [qv, colv], x16)
                return carry

            lax.fori_loop(0, d, fd, 0)

        nit = n_iters()

        def step(it, b):
            tc = tc_of(it)

            @pl.when(it + 1 < nit)
            def _():
                in_desc(tc_of(it + 1), 1 - b).start()

            in_desc(tc, b).wait()

            @pl.when(it >= 2)
            def _():
                out_desc(tc_of(it - 2), b).wait()

            transpose_tc(b)
            out_desc(tc, b).start()

        @pl.when(nit > 0)
        def _():
            in_desc(tc_of(0), 0).start()

            def body(g, carry):
                for b in range(2):
                    step(g * 2 + b, b)
                return carry

            n_pair = n_main // 2
            lax.fori_loop(0, n_pair, body, 0)
            for r in range(2):
                it = n_pair * 2 + r

                @pl.when(it < nit)
                def _(it=it, r=r):
                    step(it, r)

            for b in range(2):

                @pl.when(nit >= b + 1)
                def _(b=b):
                    out_desc(tc_of(0), b).wait()

        @pl.when(wid == NUM_WORKERS - 1)
        def _():
            pltpu.sync_copy(app_hbm, v1.at[pl.ds(0, 32), pl.ds(0, 128)])
            pltpu.sync_copy(v1.at[pl.ds(0, 32), pl.ds(0, 128)],
                            rm_hbm.at[pl.ds(v // 4 - 32, 32)])

    return ka(tT, app)


def kernel(x, table):
    b, h = x.shape
    v, d = table.shape
    app = lax.slice(table, (v - 128, 0), (v, d)).reshape(32, 128)
    rm = _relayout_table(table.T, app)
    table_rm = rm.reshape(v, d)
    out5 = _gather_rows(table_rm, x.astype(jnp.int32), b, h)
    return out5.transpose(2, 4, 0, 1, 3).reshape(b, h, EMBEDDING_DIM)


# speedup vs baseline: 1.5933x; 1.5933x over previous
import functools

import jax
import jax.numpy as jnp
from jax import lax
from jax.experimental import pallas as pl
from jax.experimental.pallas import tpu as pltpu
from jax.experimental.pallas import tpu_sc as plsc

EMBEDDING_DIM = 32
NUM_CORES = 2
NUM_SUBCORES = 16
NUM_WORKERS = NUM_CORES * NUM_SUBCORES
CHUNK_B = 16
NBUF = 2
LANES = 16


@functools.partial(jax.jit, static_argnums=(2, 3))
def _gather_rows(table, x, batch, hist):
    mesh = plsc.VectorSubcoreMesh(core_axis_name="c", subcore_axis_name="s")
    bpw = batch // NUM_WORKERS
    n_chunks = bpw // CHUNK_B
    d8 = EMBEDDING_DIM // 8
    bt = batch // 128

    @functools.partial(
        pl.kernel,
        mesh=mesh,
        out_type=jax.ShapeDtypeStruct((hist, d8, bt, 8, 128), jnp.float32),
        scratch_types=[
            pltpu.VMEM((bpw, hist), jnp.int32),
            pltpu.VMEM((CHUNK_B, hist, EMBEDDING_DIM), jnp.float32),
            pltpu.VMEM((CHUNK_B, hist, EMBEDDING_DIM), jnp.float32),
            pltpu.VMEM((hist, d8, 8, CHUNK_B), jnp.float32),
            pltpu.VMEM((hist, d8, 8, CHUNK_B), jnp.float32),
            pltpu.SemaphoreType.DMA,
            pltpu.SemaphoreType.DMA,
            pltpu.SemaphoreType.DMA,
            pltpu.SemaphoreType.DMA,
        ],
        compiler_params=pltpu.CompilerParams(use_tc_tiling_on_sc=False, needs_layout_passes=False),
    )
    def k(table_hbm, x_hbm, out_hbm, idx_v, rows0, rows1, t0, t1, g0, g1, w0, w1):
        wid = lax.axis_index("s") * NUM_CORES + lax.axis_index("c")
        base = wid * bpw
        rows = (rows0, rows1)
        tbuf = (t0, t1)
        gsem = (g0, g1)
        wsem = (w0, w1)

        pltpu.sync_copy(x_hbm.at[pl.ds(base, bpw)], idx_v)

        iot = lax.iota(jnp.int32, LANES)
        ti_lo = iot // 8
        ti_hi = ti_lo + 2
        dlo = lax.rem(iot, 8)

        def row_gather_desc(i, j, b):
            return pltpu.make_async_copy(
                table_hbm.at[idx_v.at[i * CHUNK_B + j]],
                rows[b].at[j],
                gsem[b],
            )

        def fire_gathers(i, b):
            def fj(j, c):
                row_gather_desc(i, j, b).start()
                return c

            lax.fori_loop(0, CHUNK_B, fj, 0)

        def drain_gathers(i, b):
            def fj(j, c):
                row_gather_desc(i, j, b).wait()
                return c

            lax.fori_loop(0, CHUNK_B, fj, 0)

        HU = 5

        def transpose_chunk(b):
            def fb(j, c):
                jv = jnp.full((LANES,), j, jnp.int32)

                def fh(h5, c2):
                    h0 = h5 * HU
                    for u in range(HU):
                        h = h0 + u
                        lo = rows[b][j, h, pl.ds(0, LANES)]
                        hi = rows[b][j, h, pl.ds(LANES, LANES)]
                        plsc.store_scatter(tbuf[b].at[h], [ti_lo, dlo, jv], lo)
                        plsc.store_scatter(tbuf[b].at[h], [ti_hi, dlo, jv], hi)
                    return c2

                lax.fori_loop(0, hist // HU, fh, 0)
                return c

            lax.fori_loop(0, CHUNK_B, fb, 0)

        def write_desc(i, b):
            gb = base + i * CHUNK_B
            tj = gb // 128
            bl0 = gb % 128
            return pltpu.make_async_copy(
                tbuf[b],
                out_hbm.at[pl.ds(0, hist), pl.ds(0, d8), tj, pl.ds(0, 8),
                           pl.ds(bl0, CHUNK_B)],
                wsem[b],
            )

        for b in range(NBUF):
            fire_gathers(b, b)

        def body(g, carry):
            for b in range(NBUF):
                i = g * NBUF + b
                drain_gathers(i, b)
                transpose_chunk(b)
                write_desc(i, b).start()
                write_desc(i, b).wait()
                fire_gathers(i + NBUF, b)
            return carry

        lax.fori_loop(0, n_chunks // NBUF - 1, body, 0)

        for b in range(NBUF):
            i = n_chunks - NBUF + b
            drain_gathers(i, b)
            transpose_chunk(b)
            write_desc(i, b).start()
        for b in range(NBUF):
            i = n_chunks - NBUF + b
            write_desc(i, b).wait()

    return k(table, x)


@jax.jit
def _relayout_table(tT, app):
    mesh = plsc.VectorSubcoreMesh(core_axis_name="c", subcore_axis_name="s")
    d, v = tT.shape
    BK = 512
    n_full = v // BK
    n_main = n_full // NUM_WORKERS
    n_tail = n_full % NUM_WORKERS

    @functools.partial(
        pl.kernel,
        mesh=mesh,
        out_type=jax.ShapeDtypeStruct((v // 4, 128), jnp.float32),
        scratch_types=[
            pltpu.VMEM((d, BK), jnp.float32),
            pltpu.VMEM((d, BK), jnp.float32),
            pltpu.VMEM((BK // 4, 128), jnp.float32),
            pltpu.VMEM((BK // 4, 128), jnp.float32),
            pltpu.SemaphoreType.DMA,
            pltpu.SemaphoreType.DMA,
            pltpu.SemaphoreType.DMA,
            pltpu.SemaphoreType.DMA,
        ],
        compiler_params=pltpu.CompilerParams(
            use_tc_tiling_on_sc=True, needs_layout_passes=False
        ),
    )
    def ka(tT_hbm, app_hbm, rm_hbm, v0, v1, o0, o1, gi0, gi1, go0, go1):
        wid = lax.axis_index("s") * NUM_CORES + lax.axis_index("c")
        vbuf = (v0, v1)
        obuf = (o0, o1)
        isem = (gi0, gi1)
        osem = (go0, go1)

        iot = lax.iota(jnp.int32, LANES)
        io4 = iot // 4
        iom = lax.rem(iot, 4) * 32

        def tc_of(it):
            return jnp.where(it < n_main, wid + it * NUM_WORKERS,
                             n_main * NUM_WORKERS + wid)

        def n_iters():
            return n_main + jnp.where(wid < n_tail, 1, 0)

        def in_desc(tc, b):
            return pltpu.make_async_copy(
                tT_hbm.at[pl.ds(0, d), pl.ds(tc * BK, BK)], vbuf[b], isem[b]
            )

        def out_desc(tc, b):
            return pltpu.make_async_copy(
                obuf[b], rm_hbm.at[pl.ds(tc * (BK // 4), BK // 4)], osem[b]
            )

        def transpose_tc(b):
            def fd(dd, carry):
                colv = iom + dd
                for c in range(0, BK, LANES):
                    qv = io4 + (c // 4)
                    x16 = vbuf[b][dd, pl.ds(c, LANES)]
                    plsc.store_scatter(obuf[b], [qv, colv], x16)
                return carry

            lax.fori_loop(0, d, fd, 0)

        nit = n_iters()

        def step(it, b):
            tc = tc_of(it)

            @pl.when(it + 1 < nit)
            def _():
                in_desc(tc_of(it + 1), 1 - b).start()

            in_desc(tc, b).wait()

            @pl.when(it >= 2)
            def _():
                out_desc(tc_of(it - 2), b).wait()

            transpose_tc(b)
            out_desc(tc, b).start()

        @pl.when(nit > 0)
        def _():
            in_desc(tc_of(0), 0).start()

            def body(g, carry):
                for b in range(2):
                    step(g * 2 + b, b)
                return carry

            n_pair = n_main // 2
            lax.fori_loop(0, n_pair, body, 0)
            for r in range(2):
                it = n_pair * 2 + r

                @pl.when(it < nit)
                def _(it=it, r=r):
                    step(it, r)

            for b in range(2):

                @pl.when(nit >= b + 1)
                def _(b=b):
                    out_desc(tc_of(0), b).wait()

        @pl.when(wid == NUM_WORKERS - 1)
        def _():
            pltpu.sync_copy(app_hbm, v1.at[pl.ds(0, 32), pl.ds(0, 128)])
            pltpu.sync_copy(v1.at[pl.ds(0, 32), pl.ds(0, 128)],
                            rm_hbm.at[pl.ds(v // 4 - 32, 32)])

    return ka(tT, app)


def kernel(x, table):
    b, h = x.shape
    out5 = _gather_rows(table, x.astype(jnp.int32), b, h)
    return out5.transpose(2, 4, 0, 1, 3).reshape(b, h, EMBEDDING_DIM)


# speedup vs baseline: 1.6197x; 1.0166x over previous
import functools

import jax
import jax.numpy as jnp
from jax import lax
from jax.experimental import pallas as pl
from jax.experimental.pallas import tpu as pltpu
from jax.experimental.pallas import tpu_sc as plsc

EMBEDDING_DIM = 32
NUM_CORES = 2
NUM_SUBCORES = 16
NUM_WORKERS = NUM_CORES * NUM_SUBCORES
CHUNK_B = 16
NBUF = 2
LANES = 16


@functools.partial(jax.jit, static_argnums=(2, 3))
def _gather_rows(table, x, batch, hist):
    mesh = plsc.VectorSubcoreMesh(core_axis_name="c", subcore_axis_name="s")
    bpw = batch // NUM_WORKERS
    n_chunks = bpw // CHUNK_B
    d8 = EMBEDDING_DIM // 8
    bt = batch // 128

    @functools.partial(
        pl.kernel,
        mesh=mesh,
        out_type=jax.ShapeDtypeStruct((hist, d8, bt, 8, 128), jnp.float32),
        scratch_types=[
            pltpu.VMEM((bpw, hist), jnp.int32),
            pltpu.VMEM((CHUNK_B, hist, EMBEDDING_DIM), jnp.float32),
            pltpu.VMEM((CHUNK_B, hist, EMBEDDING_DIM), jnp.float32),
            pltpu.VMEM((hist, d8, 8, CHUNK_B), jnp.float32),
            pltpu.VMEM((hist, d8, 8, CHUNK_B), jnp.float32),
            pltpu.SemaphoreType.DMA,
            pltpu.SemaphoreType.DMA,
            pltpu.SemaphoreType.DMA,
            pltpu.SemaphoreType.DMA,
        ],
        compiler_params=pltpu.CompilerParams(use_tc_tiling_on_sc=False, needs_layout_passes=False),
    )
    def k(table_hbm, x_hbm, out_hbm, idx_v, rows0, rows1, t0, t1, g0, g1, w0, w1):
        wid = lax.axis_index("s") * NUM_CORES + lax.axis_index("c")
        base = wid * bpw
        rows = (rows0, rows1)
        tbuf = (t0, t1)
        gsem = (g0, g1)
        wsem = (w0, w1)

        pltpu.sync_copy(x_hbm.at[pl.ds(base, bpw)], idx_v)

        iot = lax.iota(jnp.int32, LANES)
        ti_lo = iot // 8
        ti_hi = ti_lo + 2
        dlo = lax.rem(iot, 8)

        def row_gather_desc(i, j, b):
            return pltpu.make_async_copy(
                table_hbm.at[idx_v.at[i * CHUNK_B + j]],
                rows[b].at[j],
                gsem[b],
            )

        def fire_gathers(i, b):
            def fj(j, c):
                row_gather_desc(i, j, b).start()
                return c

            lax.fori_loop(0, CHUNK_B, fj, 0)

        def drain_gathers(i, b):
            def fj(j, c):
                row_gather_desc(i, j, b).wait()
                return c

            lax.fori_loop(0, CHUNK_B, fj, 0)

        HU = 5

        def transpose_chunk(b):
            def fb(j, c):
                jv = jnp.full((LANES,), j, jnp.int32)

                def fh(h5, c2):
                    h0 = h5 * HU
                    for u in range(HU):
                        h = h0 + u
                        lo = rows[b][j, h, pl.ds(0, LANES)]
                        hi = rows[b][j, h, pl.ds(LANES, LANES)]
                        plsc.store_scatter(tbuf[b].at[h], [ti_lo, dlo, jv], lo)
                        plsc.store_scatter(tbuf[b].at[h], [ti_hi, dlo, jv], hi)
                    return c2

                lax.fori_loop(0, hist // HU, fh, 0)
                return c

            lax.fori_loop(0, CHUNK_B, fb, 0)

        def write_desc(i, b):
            gb = base + i * CHUNK_B
            tj = gb // 128
            bl0 = gb % 128
            return pltpu.make_async_copy(
                tbuf[b],
                out_hbm.at[pl.ds(0, hist), pl.ds(0, d8), tj, pl.ds(0, 8),
                           pl.ds(bl0, CHUNK_B)],
                wsem[b],
            )

        for b in range(NBUF):
            fire_gathers(b, b)

        def body(g, carry):
            for b in range(NBUF):
                i = g * NBUF + b

                @pl.when(i >= NBUF)
                def _(i=i, b=b):
                    write_desc(i - NBUF, b).wait()

                drain_gathers(i, b)
                transpose_chunk(b)
                write_desc(i, b).start()
                fire_gathers(i + NBUF, b)
            return carry

        lax.fori_loop(0, n_chunks // NBUF - 1, body, 0)

        for b in range(NBUF):
            i = n_chunks - NBUF + b

            @pl.when(i >= NBUF)
            def _(i=i, b=b):
                write_desc(i - NBUF, b).wait()

            drain_gathers(i, b)
            transpose_chunk(b)
            write_desc(i, b).start()
        for b in range(NBUF):
            i = n_chunks - NBUF + b
            write_desc(i, b).wait()

    return k(table, x)


@jax.jit
def _relayout_table(tT, app):
    mesh = plsc.VectorSubcoreMesh(core_axis_name="c", subcore_axis_name="s")
    d, v = tT.shape
    BK = 512
    n_full = v // BK
    n_main = n_full // NUM_WORKERS
    n_tail = n_full % NUM_WORKERS

    @functools.partial(
        pl.kernel,
        mesh=mesh,
        out_type=jax.ShapeDtypeStruct((v // 4, 128), jnp.float32),
        scratch_types=[
            pltpu.VMEM((d, BK), jnp.float32),
            pltpu.VMEM((d, BK), jnp.float32),
            pltpu.VMEM((BK // 4, 128), jnp.float32),
            pltpu.VMEM((BK // 4, 128), jnp.float32),
            pltpu.SemaphoreType.DMA,
            pltpu.SemaphoreType.DMA,
            pltpu.SemaphoreType.DMA,
            pltpu.SemaphoreType.DMA,
        ],
        compiler_params=pltpu.CompilerParams(
            use_tc_tiling_on_sc=True, needs_layout_passes=False
        ),
    )
    def ka(tT_hbm, app_hbm, rm_hbm, v0, v1, o0, o1, gi0, gi1, go0, go1):
        wid = lax.axis_index("s") * NUM_CORES + lax.axis_index("c")
        vbuf = (v0, v1)
        obuf = (o0, o1)
        isem = (gi0, gi1)
        osem = (go0, go1)

        iot = lax.iota(jnp.int32, LANES)
        io4 = iot // 4
        iom = lax.rem(iot, 4) * 32

        def tc_of(it):
            return jnp.where(it < n_main, wid + it * NUM_WORKERS,
                             n_main * NUM_WORKERS + wid)

        def n_iters():
            return n_main + jnp.where(wid < n_tail, 1, 0)

        def in_desc(tc, b):
            return pltpu.make_async_copy(
                tT_hbm.at[pl.ds(0, d), pl.ds(tc * BK, BK)], vbuf[b], isem[b]
            )

        def out_desc(tc, b):
            return pltpu.make_async_copy(
                obuf[b], rm_hbm.at[pl.ds(tc * (BK // 4), BK // 4)], osem[b]
            )

        def transpose_tc(b):
            def fd(dd, carry):
                colv = iom + dd
                for c in range(0, BK, LANES):
                    qv = io4 + (c // 4)
                    x16 = vbuf[b][dd, pl.ds(c, LANES)]
                    plsc.store_scatter(obuf[b], [qv, colv], x16)
                return carry

            lax.fori_loop(0, d, fd, 0)

        nit = n_iters()

        def step(it, b):
            tc = tc_of(it)

            @pl.when(it + 1 < nit)
            def _():
                in_desc(tc_of(it + 1), 1 - b).start()

            in_desc(tc, b).wait()

            @pl.when(it >= 2)
            def _():
                out_desc(tc_of(it - 2), b).wait()

            transpose_tc(b)
            out_desc(tc, b).start()

        @pl.when(nit > 0)
        def _():
            in_desc(tc_of(0), 0).start()

            def body(g, carry):
                for b in range(2):
                    step(g * 2 + b, b)
                return carry

            n_pair = n_main // 2
            lax.fori_loop(0, n_pair, body, 0)
            for r in range(2):
                it = n_pair * 2 + r

                @pl.when(it < nit)
                def _(it=it, r=r):
                    step(it, r)

            for b in range(2):

                @pl.when(nit >= b + 1)
                def _(b=b):
                    out_desc(tc_of(0), b).wait()

        @pl.when(wid == NUM_WORKERS - 1)
        def _():
            pltpu.sync_copy(app_hbm, v1.at[pl.ds(0, 32), pl.ds(0, 128)])
            pltpu.sync_copy(v1.at[pl.ds(0, 32), pl.ds(0, 128)],
                            rm_hbm.at[pl.ds(v // 4 - 32, 32)])

    return ka(tT, app)


def kernel(x, table):
    b, h = x.shape
    out5 = _gather_rows(table, x.astype(jnp.int32), b, h)
    return out5.transpose(2, 4, 0, 1, 3).reshape(b, h, EMBEDDING_DIM)
